# baseline (speedup 1.0000x reference)
import functools

import jax
import jax.numpy as jnp
from jax import lax
from jax.experimental import pallas as pl
from jax.experimental.pallas import tpu as pltpu
from jax.experimental.pallas import tpu_sc as plsc

EPS_ = 0.0001
LN_EPS = 1e-5

NC = 2
NS = 16
K = 80


def _pre_h_body(x_ref, fcw_ref, fcb_ref, h_ref):
    h_ref[...] = jnp.dot(x_ref[...], fcw_ref[...],
                         preferred_element_type=jnp.float32) + fcb_ref[...]


def _pre_rg_body(x_ref, ratew_ref, robw_ref, robb_ref, rate_ref, gamma_ref):
    x = x_ref[...]
    z = jnp.dot(x, ratew_ref[...], preferred_element_type=jnp.float32)
    rate_ref[...] = jax.nn.softplus(z) + EPS_
    gamma_ref[...] = jnp.dot(x, robw_ref[...],
                             preferred_element_type=jnp.float32) + robb_ref[...]


def _pre_h(x, fcw_t, fcb, bn):
    n, d = x.shape
    blk_x = pl.BlockSpec((bn, d), lambda i: (i, 0))
    blk_w = pl.BlockSpec((d, d), lambda i: (0, 0))
    blk_b = pl.BlockSpec((1, d), lambda i: (0, 0))
    return pl.pallas_call(
        _pre_h_body,
        grid=(n // bn,),
        in_specs=[blk_x, blk_w, blk_b],
        out_specs=blk_x,
        out_shape=jax.ShapeDtypeStruct((n, d), jnp.float32),
    )(x, fcw_t, fcb)


def _pre_rg(x, ratew_t, robw_t, robb, bn):
    n, d = x.shape
    blk_x = pl.BlockSpec((bn, d), lambda i: (i, 0))
    blk_w = pl.BlockSpec((d, d), lambda i: (0, 0))
    blk_b = pl.BlockSpec((1, d), lambda i: (0, 0))
    return pl.pallas_call(
        _pre_rg_body,
        grid=(n // bn,),
        in_specs=[blk_x, blk_w, blk_w, blk_b],
        out_specs=[blk_x, blk_x],
        out_shape=[jax.ShapeDtypeStruct((n, d), jnp.float32)] * 2,
    )(x, ratew_t, robw_t, robb)


def _make_agg(n, d, e, n_pad):
    nw = NC * NS
    ep = e // nw
    ch = ep // K
    rp = n_pad // NS
    assert ep * nw == e and ch * K == ep and rp % 16 == 0 and ep % 8 == 0

    mesh = plsc.VectorSubcoreMesh(core_axis_name="c", subcore_axis_name="s",
                                  num_cores=NC, num_subcores=NS)

    GS = 4
    IS = 8

    scratch = (
        [pltpu.VMEM((K, d), jnp.float32)] * GS
        + [pltpu.VMEM((K,), jnp.int32)] * IS
        + [pltpu.VMEM((K,), jnp.int32)] * IS
        + [pltpu.VMEM((K,), jnp.float32),
           pltpu.VMEM((rp,), jnp.float32)]
        + [pltpu.SemaphoreType.DMA] * (GS * 3 + IS)
        + [pltpu.VMEM_SHARED((n_pad, d), jnp.float32),
           pltpu.VMEM_SHARED((n_pad,), jnp.float32)]
    )

    @functools.partial(
        pl.kernel,
        out_type=(jax.ShapeDtypeStruct((NC * n_pad, d), jnp.float32),
                  jax.ShapeDtypeStruct((NC * n_pad,), jnp.float32)),
        mesh=mesh,
        scratch_types=scratch,
    )
    def agg(h_hbm, row_hbm, col_hbm, s_out, cnt_out, *sc):
        bufs = list(sc[0:GS])
        cidxs = list(sc[GS:GS + IS])
        ridxs = list(sc[GS + IS:GS + 2 * IS])
        ones_v, cstage_v = sc[GS + 2 * IS], sc[GS + 2 * IS + 1]
        p = GS + 2 * IS + 2
        semG = list(sc[p:p + GS])
        semS = list(sc[p + GS:p + 2 * GS])
        semC = list(sc[p + 2 * GS:p + 3 * GS])
        semI = list(sc[p + 3 * GS:p + 3 * GS + IS])
        acc_sh, cnt_sh = sc[p + 3 * GS + IS], sc[p + 3 * GS + IS + 1]

        c = lax.axis_index("c")
        s = lax.axis_index("s")
        wid = s * NC + c
        base = wid * ep

        def iload(j, isl):
            pltpu.async_copy(col_hbm.at[pl.ds(base + j * K, K)],
                             cidxs[isl], semI[isl])
            pltpu.async_copy(row_hbm.at[pl.ds(base + j * K, K)],
                             ridxs[isl], semI[isl])

        def iwait(j, isl):
            pltpu.make_async_copy(col_hbm.at[pl.ds(base + j * K, K)],
                                  cidxs[isl], semI[isl]).wait()
            pltpu.make_async_copy(row_hbm.at[pl.ds(base + j * K, K)],
                                  ridxs[isl], semI[isl]).wait()

        def gstart(isl, gs):
            pltpu.async_copy(h_hbm.at[cidxs[isl]], bufs[gs], semG[gs])

        def gwait(isl, gs):
            pltpu.make_async_copy(h_hbm.at[cidxs[isl]], bufs[gs],
                                  semG[gs]).wait()

        def sstart(isl, gs):
            pltpu.async_copy(bufs[gs], acc_sh.at[ridxs[isl]], semS[gs],
                             add=True)
            pltpu.async_copy(ones_v, cnt_sh.at[ridxs[isl]], semC[gs],
                             add=True)

        def swait(isl, gs):
            pltpu.make_async_copy(bufs[gs], acc_sh.at[ridxs[isl]],
                                  semS[gs]).wait()
            pltpu.make_async_copy(ones_v, cnt_sh.at[ridxs[isl]],
                                  semC[gs]).wait()

        iload(0, 0)
        iload(1, 1)
        iload(2, 2)

        @pl.loop(0, 16)
        def _zr(r):
            @pl.loop(0, d // 16)
            def _zc(j):
                bufs[0][r, pl.ds(j * 16, 16)] = jnp.zeros((16,), jnp.float32)

        @pl.loop(0, rp // 16)
        def _z(j):
            cstage_v[pl.ds(j * 16, 16)] = jnp.zeros((16,), jnp.float32)

        @pl.loop(0, K // 16)
        def _o(j):
            ones_v[pl.ds(j * 16, 16)] = jnp.full((16,), 1.0, jnp.float32)

        @pl.loop(0, rp // 16)
        def _za(j):
            pltpu.sync_copy(bufs[0].at[pl.ds(0, 16)],
                            acc_sh.at[pl.ds(s * rp + j * 16, 16)])

        pltpu.sync_copy(cstage_v, cnt_sh.at[pl.ds(s * rp, rp)])

        iwait(0, 0)
        gstart(0, 0)
        iwait(1, 1)
        gstart(1, 1)
        plsc.subcore_barrier()

        n_outer = -(-(ch + 2) // IS)

        @pl.loop(0, n_outer * IS, step=IS)
        def _outer(i):
            for b in range(IS):
                j = i + b

                jj = j - 2
                if b >= 2:
                    cond_a = jj < ch
                else:
                    cond_a = jnp.logical_and(jj >= 0, jj < ch)

                @pl.when(cond_a)
                def _a(jj=jj, b=b):
                    swait((b - 2) % IS, (b - 2) % GS)

                @pl.when(j + 3 < ch)
                def _b(j=j, b=b):
                    iload(j + 3, (b + 3) % IS)

                @pl.when(j + 2 < ch)
                def _c(j=j, b=b):
                    iwait(j + 2, (b + 2) % IS)
                    gstart((b + 2) % IS, (b + 2) % GS)

                @pl.when(j < ch)
                def _d(j=j, b=b):
                    gwait(b % IS, b % GS)
                    sstart(b % IS, b % GS)

        plsc.subcore_barrier()

        pltpu.sync_copy(acc_sh.at[pl.ds(s * rp, rp)],
                        s_out.at[pl.ds(c * n_pad + s * rp, rp)])
        pltpu.sync_copy(cnt_sh.at[pl.ds(s * rp, rp)], cstage_v)
        pltpu.sync_copy(cstage_v, cnt_out.at[pl.ds(c * n_pad + s * rp, rp)])

    return agg


def _post_body(h_ref, rate_ref, gamma_ref, deg_ref, s0_ref, s1_ref,
               c0_ref, c1_ref, lnw_ref, lnb_ref, out_ref):
    cnt = c0_ref[0] + c1_ref[0]
    agg = cnt * h_ref[...] + s0_ref[0] + s1_ref[0]
    r = rate_ref[...]
    out = (r * agg + gamma_ref[...]) / (1.0 + r * deg_ref[...] + EPS_)
    mean = jnp.mean(out, axis=-1, keepdims=True)
    cen = out - mean
    var = jnp.mean(cen * cen, axis=-1, keepdims=True)
    out_ref[...] = cen / jnp.sqrt(var + LN_EPS) * lnw_ref[...] + lnb_ref[...]


def _post(h, rate, gamma, deg2, s_part, cnt_part, lnw, lnb, bn):
    n, d = h.shape
    nb = n // bn
    grid = (nb,)
    blk = pl.BlockSpec((bn, d), lambda i: (i, 0))
    blk1 = pl.BlockSpec((bn, 1), lambda i: (i, 0))
    blk_s0 = pl.BlockSpec((1, bn, d), lambda i: (0, i, 0))
    blk_s1 = pl.BlockSpec((1, bn, d), lambda i: (1, i, 0))
    blk_c0 = pl.BlockSpec((1, bn, 1), lambda i: (0, i, 0))
    blk_c1 = pl.BlockSpec((1, bn, 1), lambda i: (1, i, 0))
    blk_ln = pl.BlockSpec((1, d), lambda i: (0, 0))
    return pl.pallas_call(
        _post_body,
        grid=grid,
        in_specs=[blk, blk, blk, blk1, blk_s0, blk_s1, blk_c0, blk_c1,
                  blk_ln, blk_ln],
        out_specs=blk,
        out_shape=jax.ShapeDtypeStruct((n, d), jnp.float32),
    )(h, rate, gamma, deg2, s_part, s_part, cnt_part, cnt_part, lnw, lnb)


def kernel(x, edge_index, degree, fc_w, fc_b, rate_w, rob_w, rob_b,
           ln_w, ln_b):
    n, d = x.shape
    e = edge_index.shape[1]
    bn = 2000
    assert n % bn == 0

    row = edge_index[0]
    col = edge_index[1]

    h = _pre_h(x, fc_w.T, fc_b.reshape(1, d), bn)

    n_pad = -(-n // (NS * 16)) * (NS * 16)
    s_part, cnt_part = _make_agg(n, d, e, n_pad)(h, row, col)

    rate, gamma = _pre_rg(x, rate_w.T, rob_w.T, rob_b.reshape(1, d), bn)
    s_part = s_part.reshape(NC, n_pad, d)
    cnt_part = cnt_part.reshape(NC, n_pad, 1)

    return _post(h, rate, gamma, degree.reshape(n, 1), s_part, cnt_part,
                 ln_w.reshape(1, d), ln_b.reshape(1, d), bn)


# speedup vs baseline: 1.0142x; 1.0142x over previous
import functools

import jax
import jax.numpy as jnp
from jax import lax
from jax.experimental import pallas as pl
from jax.experimental.pallas import tpu as pltpu
from jax.experimental.pallas import tpu_sc as plsc

EPS_ = 0.0001
LN_EPS = 1e-5

NC = 2
NS = 16
K = 80


def _pre_h_body(x_ref, fcw_ref, fcb_ref, h_ref):
    h_ref[...] = jnp.dot(x_ref[...], fcw_ref[...],
                         preferred_element_type=jnp.float32) + fcb_ref[...]


def _pre_rg_body(x_ref, ratew_ref, robw_ref, robb_ref, rate_ref, gamma_ref):
    x = x_ref[...]
    z = jnp.dot(x, ratew_ref[...], preferred_element_type=jnp.float32)
    rate_ref[...] = jax.nn.softplus(z) + EPS_
    gamma_ref[...] = jnp.dot(x, robw_ref[...],
                             preferred_element_type=jnp.float32) + robb_ref[...]


def _pre_h(x, fcw_t, fcb, bn):
    n, d = x.shape
    blk_x = pl.BlockSpec((bn, d), lambda i: (i, 0))
    blk_w = pl.BlockSpec((d, d), lambda i: (0, 0))
    blk_b = pl.BlockSpec((1, d), lambda i: (0, 0))
    return pl.pallas_call(
        _pre_h_body,
        grid=(n // bn,),
        in_specs=[blk_x, blk_w, blk_b],
        out_specs=blk_x,
        out_shape=jax.ShapeDtypeStruct((n, d), jnp.float32),
    )(x, fcw_t, fcb)


def _pre_rg(x, ratew_t, robw_t, robb, bn):
    n, d = x.shape
    blk_x = pl.BlockSpec((bn, d), lambda i: (i, 0))
    blk_w = pl.BlockSpec((d, d), lambda i: (0, 0))
    blk_b = pl.BlockSpec((1, d), lambda i: (0, 0))
    return pl.pallas_call(
        _pre_rg_body,
        grid=(n // bn,),
        in_specs=[blk_x, blk_w, blk_w, blk_b],
        out_specs=[blk_x, blk_x],
        out_shape=[jax.ShapeDtypeStruct((n, d), jnp.float32)] * 2,
    )(x, ratew_t, robw_t, robb)


def _make_agg(n, d, e, n_pad):
    nw = NC * NS
    ep = e // nw
    ch = ep // K
    rp = n_pad // NS
    assert ep * nw == e and ch * K == ep and rp % 16 == 0 and ep % 8 == 0

    mesh = plsc.VectorSubcoreMesh(core_axis_name="c", subcore_axis_name="s",
                                  num_cores=NC, num_subcores=NS)

    GS = 4
    IS = 8

    scratch = (
        [pltpu.VMEM((K, d), jnp.float32)] * GS
        + [pltpu.VMEM((K,), jnp.int32)] * IS
        + [pltpu.VMEM((K,), jnp.int32)] * IS
        + [pltpu.VMEM((K,), jnp.float32),
           pltpu.VMEM((rp,), jnp.float32)]
        + [pltpu.SemaphoreType.DMA] * (GS * 3 + IS)
        + [pltpu.VMEM_SHARED((n_pad, d), jnp.float32),
           pltpu.VMEM_SHARED((n_pad,), jnp.float32)]
    )

    @functools.partial(
        pl.kernel,
        out_type=(jax.ShapeDtypeStruct((NC * n_pad, d), jnp.float32),
                  jax.ShapeDtypeStruct((NC * n_pad,), jnp.float32)),
        mesh=mesh,
        scratch_types=scratch,
    )
    def agg(h_hbm, row_hbm, col_hbm, s_out, cnt_out, *sc):
        bufs = list(sc[0:GS])
        cidxs = list(sc[GS:GS + IS])
        ridxs = list(sc[GS + IS:GS + 2 * IS])
        ones_v, cstage_v = sc[GS + 2 * IS], sc[GS + 2 * IS + 1]
        p = GS + 2 * IS + 2
        semG = list(sc[p:p + GS])
        semS = list(sc[p + GS:p + 2 * GS])
        semC = list(sc[p + 2 * GS:p + 3 * GS])
        semI = list(sc[p + 3 * GS:p + 3 * GS + IS])
        acc_sh, cnt_sh = sc[p + 3 * GS + IS], sc[p + 3 * GS + IS + 1]

        c = lax.axis_index("c")
        s = lax.axis_index("s")
        wid = s * NC + c
        base = wid * ep

        def iload(j, isl):
            pltpu.async_copy(col_hbm.at[pl.ds(base + j * K, K)],
                             cidxs[isl], semI[isl])
            pltpu.async_copy(row_hbm.at[pl.ds(base + j * K, K)],
                             ridxs[isl], semI[isl])

        def iwait(j, isl):
            pltpu.make_async_copy(col_hbm.at[pl.ds(base + j * K, K)],
                                  cidxs[isl], semI[isl]).wait()
            pltpu.make_async_copy(row_hbm.at[pl.ds(base + j * K, K)],
                                  ridxs[isl], semI[isl]).wait()

        def gstart(isl, gs):
            pltpu.async_copy(h_hbm.at[cidxs[isl]], bufs[gs], semG[gs])

        def gwait(isl, gs):
            pltpu.make_async_copy(h_hbm.at[cidxs[isl]], bufs[gs],
                                  semG[gs]).wait()

        def sstart(isl, gs):
            pltpu.async_copy(bufs[gs], acc_sh.at[ridxs[isl]], semS[gs],
                             add=True)
            pltpu.async_copy(ones_v, cnt_sh.at[ridxs[isl]], semC[gs],
                             add=True)

        def swait(isl, gs):
            pltpu.make_async_copy(bufs[gs], acc_sh.at[ridxs[isl]],
                                  semS[gs]).wait()
            pltpu.make_async_copy(ones_v, cnt_sh.at[ridxs[isl]],
                                  semC[gs]).wait()

        iload(0, 0)
        iload(1, 1)
        iload(2, 2)

        @pl.loop(0, 16)
        def _zr(r):
            @pl.loop(0, d // 16)
            def _zc(j):
                bufs[0][r, pl.ds(j * 16, 16)] = jnp.zeros((16,), jnp.float32)

        @pl.loop(0, rp // 16)
        def _z(j):
            cstage_v[pl.ds(j * 16, 16)] = jnp.zeros((16,), jnp.float32)

        @pl.loop(0, K // 16)
        def _o(j):
            ones_v[pl.ds(j * 16, 16)] = jnp.full((16,), 1.0, jnp.float32)

        @pl.loop(0, rp // 16)
        def _za(j):
            pltpu.sync_copy(bufs[0].at[pl.ds(0, 16)],
                            acc_sh.at[pl.ds(s * rp + j * 16, 16)])

        pltpu.sync_copy(cstage_v, cnt_sh.at[pl.ds(s * rp, rp)])

        iwait(0, 0)
        gstart(0, 0)
        iwait(1, 1)
        gstart(1, 1)
        plsc.subcore_barrier()

        n_outer = -(-(ch + 2) // IS)

        @pl.loop(0, n_outer * IS, step=IS)
        def _outer(i):
            for b in range(IS):
                j = i + b

                jj = j - 2
                if b >= 2:
                    cond_a = jj < ch
                else:
                    cond_a = jnp.logical_and(jj >= 0, jj < ch)

                @pl.when(cond_a)
                def _a(jj=jj, b=b):
                    swait((b - 2) % IS, (b - 2) % GS)

                @pl.when(j + 3 < ch)
                def _b(j=j, b=b):
                    iload(j + 3, (b + 3) % IS)

                @pl.when(j + 2 < ch)
                def _c(j=j, b=b):
                    iwait(j + 2, (b + 2) % IS)
                    gstart((b + 2) % IS, (b + 2) % GS)

                @pl.when(j < ch)
                def _d(j=j, b=b):
                    gwait(b % IS, b % GS)
                    sstart(b % IS, b % GS)

        plsc.subcore_barrier()

        pltpu.sync_copy(acc_sh.at[pl.ds(s * rp, rp)],
                        s_out.at[pl.ds(c * n_pad + s * rp, rp)])
        pltpu.sync_copy(cnt_sh.at[pl.ds(s * rp, rp)], cstage_v)
        pltpu.sync_copy(cstage_v, cnt_out.at[pl.ds(c * n_pad + s * rp, rp)])

    return agg


def _post_body(h_ref, x_ref, ratew_ref, robw_ref, robb_ref, deg_ref,
               s0_ref, s1_ref, c0_ref, c1_ref, lnw_ref, lnb_ref, out_ref):
    x = x_ref[...]
    z = jnp.dot(x, ratew_ref[...], preferred_element_type=jnp.float32)
    r = jax.nn.softplus(z) + EPS_
    gamma = jnp.dot(x, robw_ref[...],
                    preferred_element_type=jnp.float32) + robb_ref[...]
    cnt = c0_ref[0] + c1_ref[0]
    agg = cnt * h_ref[...] + s0_ref[0] + s1_ref[0]
    out = (r * agg + gamma) / (1.0 + r * deg_ref[...] + EPS_)
    mean = jnp.mean(out, axis=-1, keepdims=True)
    cen = out - mean
    var = jnp.mean(cen * cen, axis=-1, keepdims=True)
    out_ref[...] = cen / jnp.sqrt(var + LN_EPS) * lnw_ref[...] + lnb_ref[...]


def _post(h, x, ratew_t, robw_t, robb, deg2, s_part, cnt_part, lnw, lnb, bn):
    n, d = h.shape
    nb = n // bn
    grid = (nb,)
    blk = pl.BlockSpec((bn, d), lambda i: (i, 0))
    blk_w = pl.BlockSpec((d, d), lambda i: (0, 0))
    blk1 = pl.BlockSpec((bn, 1), lambda i: (i, 0))
    blk_s0 = pl.BlockSpec((1, bn, d), lambda i: (0, i, 0))
    blk_s1 = pl.BlockSpec((1, bn, d), lambda i: (1, i, 0))
    blk_c0 = pl.BlockSpec((1, bn, 1), lambda i: (0, i, 0))
    blk_c1 = pl.BlockSpec((1, bn, 1), lambda i: (1, i, 0))
    blk_ln = pl.BlockSpec((1, d), lambda i: (0, 0))
    return pl.pallas_call(
        _post_body,
        grid=grid,
        in_specs=[blk, blk, blk_w, blk_w, blk_ln, blk1, blk_s0, blk_s1,
                  blk_c0, blk_c1, blk_ln, blk_ln],
        out_specs=blk,
        out_shape=jax.ShapeDtypeStruct((n, d), jnp.float32),
    )(h, x, ratew_t, robw_t, robb, deg2, s_part, s_part, cnt_part,
      cnt_part, lnw, lnb)


def kernel(x, edge_index, degree, fc_w, fc_b, rate_w, rob_w, rob_b,
           ln_w, ln_b):
    n, d = x.shape
    e = edge_index.shape[1]
    bn = 2000
    assert n % bn == 0

    row = edge_index[0]
    col = edge_index[1]

    h = _pre_h(x, fc_w.T, fc_b.reshape(1, d), bn)

    n_pad = -(-n // (NS * 16)) * (NS * 16)
    s_part, cnt_part = _make_agg(n, d, e, n_pad)(h, row, col)

    s_part = s_part.reshape(NC, n_pad, d)
    cnt_part = cnt_part.reshape(NC, n_pad, 1)

    return _post(h, x, rate_w.T, rob_w.T, rob_b.reshape(1, d),
                 degree.reshape(n, 1), s_part, cnt_part,
                 ln_w.reshape(1, d), ln_b.reshape(1, d), bn)


# speedup vs baseline: 1.0150x; 1.0008x over previous
import functools

import jax
import jax.numpy as jnp
from jax import lax
from jax.experimental import pallas as pl
from jax.experimental.pallas import tpu as pltpu
from jax.experimental.pallas import tpu_sc as plsc

EPS_ = 0.0001
LN_EPS = 1e-5

NC = 2
NS = 16
K = 80


def _pre_h_body(x_ref, fcw_ref, fcb_ref, h_ref):
    h_ref[...] = jnp.dot(x_ref[...], fcw_ref[...],
                         preferred_element_type=jnp.float32) + fcb_ref[...]


def _pre_h(x, fcw_t, fcb, bn):
    n, d = x.shape
    blk_x = pl.BlockSpec((bn, d), lambda i: (i, 0))
    blk_w = pl.BlockSpec((d, d), lambda i: (0, 0))
    blk_b = pl.BlockSpec((1, d), lambda i: (0, 0))
    return pl.pallas_call(
        _pre_h_body,
        grid=(n // bn,),
        in_specs=[blk_x, blk_w, blk_b],
        out_specs=blk_x,
        out_shape=jax.ShapeDtypeStruct((n, d), jnp.float32),
    )(x, fcw_t, fcb)


def _make_agg(n, d, e, n_pad):
    nw = NC * NS
    ep = e // nw
    ch = ep // K
    rp = n_pad // NS
    assert ep * nw == e and ch * K == ep and rp % 16 == 0 and ep % 8 == 0

    mesh = plsc.VectorSubcoreMesh(core_axis_name="c", subcore_axis_name="s",
                                  num_cores=NC, num_subcores=NS)

    GS = 4
    IS = 8

    scratch = (
        [pltpu.VMEM((K, d), jnp.float32)] * GS
        + [pltpu.VMEM((K,), jnp.int32)] * IS
        + [pltpu.VMEM((K,), jnp.int32)] * IS
        + [pltpu.VMEM((K,), jnp.float32),
           pltpu.VMEM((rp,), jnp.float32)]
        + [pltpu.SemaphoreType.DMA] * (GS * 3 + IS)
        + [pltpu.VMEM_SHARED((n_pad, d), jnp.float32),
           pltpu.VMEM_SHARED((n_pad,), jnp.float32)]
    )

    @functools.partial(
        pl.kernel,
        out_type=(jax.ShapeDtypeStruct((NC * n_pad, d), jnp.float32),
                  jax.ShapeDtypeStruct((NC * n_pad,), jnp.float32)),
        mesh=mesh,
        scratch_types=scratch,
    )
    def agg(h_hbm, row_hbm, col_hbm, s_out, cnt_out, *sc):
        bufs = list(sc[0:GS])
        cidxs = list(sc[GS:GS + IS])
        ridxs = list(sc[GS + IS:GS + 2 * IS])
        ones_v, cstage_v = sc[GS + 2 * IS], sc[GS + 2 * IS + 1]
        p = GS + 2 * IS + 2
        semG = list(sc[p:p + GS])
        semS = list(sc[p + GS:p + 2 * GS])
        semC = list(sc[p + 2 * GS:p + 3 * GS])
        semI = list(sc[p + 3 * GS:p + 3 * GS + IS])
        acc_sh, cnt_sh = sc[p + 3 * GS + IS], sc[p + 3 * GS + IS + 1]

        c = lax.axis_index("c")
        s = lax.axis_index("s")
        wid = s * NC + c
        base = wid * ep

        def iload(j, isl):
            pltpu.async_copy(col_hbm.at[pl.ds(base + j * K, K)],
                             cidxs[isl], semI[isl])
            pltpu.async_copy(row_hbm.at[pl.ds(base + j * K, K)],
                             ridxs[isl], semI[isl])

        def iwait(j, isl):
            pltpu.make_async_copy(col_hbm.at[pl.ds(base + j * K, K)],
                                  cidxs[isl], semI[isl]).wait()
            pltpu.make_async_copy(row_hbm.at[pl.ds(base + j * K, K)],
                                  ridxs[isl], semI[isl]).wait()

        def gstart(isl, gs):
            pltpu.async_copy(h_hbm.at[cidxs[isl]], bufs[gs], semG[gs])

        def gwait(isl, gs):
            pltpu.make_async_copy(h_hbm.at[cidxs[isl]], bufs[gs],
                                  semG[gs]).wait()

        def sstart(isl, gs):
            pltpu.async_copy(bufs[gs], acc_sh.at[ridxs[isl]], semS[gs],
                             add=True)
            pltpu.async_copy(ones_v, cnt_sh.at[ridxs[isl]], semC[gs],
                             add=True)

        def swait(isl, gs):
            pltpu.make_async_copy(bufs[gs], acc_sh.at[ridxs[isl]],
                                  semS[gs]).wait()
            pltpu.make_async_copy(ones_v, cnt_sh.at[ridxs[isl]],
                                  semC[gs]).wait()

        iload(0, 0)
        iload(1, 1)
        iload(2, 2)

        @pl.loop(0, 16)
        def _zr(r):
            @pl.loop(0, d // 16)
            def _zc(j):
                bufs[0][r, pl.ds(j * 16, 16)] = jnp.zeros((16,), jnp.float32)

        @pl.loop(0, rp // 16)
        def _z(j):
            cstage_v[pl.ds(j * 16, 16)] = jnp.zeros((16,), jnp.float32)

        @pl.loop(0, K // 16)
        def _o(j):
            ones_v[pl.ds(j * 16, 16)] = jnp.full((16,), 1.0, jnp.float32)

        @pl.loop(0, rp // 16)
        def _za(j):
            pltpu.sync_copy(bufs[0].at[pl.ds(0, 16)],
                            acc_sh.at[pl.ds(s * rp + j * 16, 16)])

        pltpu.sync_copy(cstage_v, cnt_sh.at[pl.ds(s * rp, rp)])

        iwait(0, 0)
        gstart(0, 0)
        iwait(1, 1)
        gstart(1, 1)
        plsc.subcore_barrier()

        n_outer = -(-(ch + 2) // IS)

        @pl.loop(0, n_outer * IS, step=IS)
        def _outer(i):
            for b in range(IS):
                j = i + b

                jj = j - 2
                if b >= 2:
                    cond_a = jj < ch
                else:
                    cond_a = jnp.logical_and(jj >= 0, jj < ch)

                @pl.when(cond_a)
                def _a(jj=jj, b=b):
                    swait((b - 2) % IS, (b - 2) % GS)

                @pl.when(j + 3 < ch)
                def _b(j=j, b=b):
                    iload(j + 3, (b + 3) % IS)

                @pl.when(j + 2 < ch)
                def _c(j=j, b=b):
                    iwait(j + 2, (b + 2) % IS)
                    gstart((b + 2) % IS, (b + 2) % GS)

                @pl.when(j < ch)
                def _d(j=j, b=b):
                    gwait(b % IS, b % GS)
                    sstart(b % IS, b % GS)

        plsc.subcore_barrier()

        pltpu.sync_copy(acc_sh.at[pl.ds(s * rp, rp)],
                        s_out.at[pl.ds(c * n_pad + s * rp, rp)])
        pltpu.sync_copy(cnt_sh.at[pl.ds(s * rp, rp)], cstage_v)
        pltpu.sync_copy(cstage_v, cnt_out.at[pl.ds(c * n_pad + s * rp, rp)])

    return agg


def _post_body(h_ref, x_ref, ratew_ref, robw_ref, robb_ref, deg_ref,
               s0_ref, s1_ref, c0_ref, c1_ref, lnw_ref, lnb_ref, out_ref):
    x = x_ref[...]
    z = jnp.dot(x, ratew_ref[...], preferred_element_type=jnp.float32)
    r = jax.nn.softplus(z) + EPS_
    gamma = jnp.dot(x, robw_ref[...],
                    preferred_element_type=jnp.float32) + robb_ref[...]
    cnt = c0_ref[0] + c1_ref[0]
    agg = cnt * h_ref[...] + s0_ref[0] + s1_ref[0]
    out = (r * agg + gamma) / (1.0 + r * deg_ref[...] + EPS_)
    mean = jnp.mean(out, axis=-1, keepdims=True)
    cen = out - mean
    var = jnp.mean(cen * cen, axis=-1, keepdims=True)
    out_ref[...] = cen / jnp.sqrt(var + LN_EPS) * lnw_ref[...] + lnb_ref[...]


def _post(h, x, ratew_t, robw_t, robb, deg2, s_part, cnt_part, lnw, lnb, bn):
    n, d = h.shape
    nb = n // bn
    grid = (nb,)
    blk = pl.BlockSpec((bn, d), lambda i: (i, 0))
    blk_w = pl.BlockSpec((d, d), lambda i: (0, 0))
    blk1 = pl.BlockSpec((bn, 1), lambda i: (i, 0))
    blk_s0 = pl.BlockSpec((1, bn, d), lambda i: (0, i, 0))
    blk_s1 = pl.BlockSpec((1, bn, d), lambda i: (1, i, 0))
    blk_c0 = pl.BlockSpec((1, bn, 1), lambda i: (0, i, 0))
    blk_c1 = pl.BlockSpec((1, bn, 1), lambda i: (1, i, 0))
    blk_ln = pl.BlockSpec((1, d), lambda i: (0, 0))
    return pl.pallas_call(
        _post_body,
        grid=grid,
        in_specs=[blk, blk, blk_w, blk_w, blk_ln, blk1, blk_s0, blk_s1,
                  blk_c0, blk_c1, blk_ln, blk_ln],
        out_specs=blk,
        out_shape=jax.ShapeDtypeStruct((n, d), jnp.float32),
    )(h, x, ratew_t, robw_t, robb, deg2, s_part, s_part, cnt_part,
      cnt_part, lnw, lnb)


def kernel(x, edge_index, degree, fc_w, fc_b, rate_w, rob_w, rob_b,
           ln_w, ln_b):
    n, d = x.shape
    e = edge_index.shape[1]
    bn = 2000
    assert n % bn == 0

    row = edge_index[0]
    col = edge_index[1]

    h = _pre_h(x, fc_w.T, fc_b.reshape(1, d), bn)

    n_pad = -(-n // (NS * 16)) * (NS * 16)
    s_part, cnt_part = _make_agg(n, d, e, n_pad)(h, row, col)

    s_part = s_part.reshape(NC, n_pad, d)
    cnt_part = cnt_part.reshape(NC, n_pad, 1)

    return _post(h, x, rate_w.T, rob_w.T, rob_b.reshape(1, d),
                 degree.reshape(n, 1), s_part, cnt_part,
                 ln_w.reshape(1, d), ln_b.reshape(1, d), bn)
